# baseline (speedup 1.0000x reference)
import functools

import jax
import jax.numpy as jnp
from jax import lax
from jax.experimental import pallas as pl
from jax.experimental.pallas import tpu as pltpu
from jax.experimental.pallas import tpu_sc as plsc

D_MODEL = 64
SCALE = 8.0
NC, NS = 2, 16
NW = NC * NS
C = 128


def _build_sc_kernel(S, B):
    R = S * B
    n_bt = B // C
    rows_per_w = R // NW
    T = rows_per_w // C
    assert T % 4 == 0 and T >= 8
    groups = T // 4
    mesh = plsc.VectorSubcoreMesh(core_axis_name="c", subcore_axis_name="s")

    @functools.partial(
        pl.kernel,
        out_type=jax.ShapeDtypeStruct((S, D_MODEL // 8, n_bt, 8, C),
                                      jnp.float32),
        mesh=mesh,
        compiler_params=pltpu.CompilerParams(use_tc_tiling_on_sc=False,
                                             needs_layout_passes=False),
        scratch_types=[
            pltpu.VMEM((T, C), jnp.int32),
            pltpu.VMEM((C, D_MODEL), jnp.float32),
            pltpu.VMEM((C, D_MODEL), jnp.float32),
            pltpu.VMEM((C, D_MODEL), jnp.float32),
            pltpu.VMEM((C, D_MODEL), jnp.float32),
            pltpu.VMEM((D_MODEL // 8, 8, C), jnp.float32),
            pltpu.VMEM((D_MODEL // 8, 8, C), jnp.float32),
            pltpu.VMEM((D_MODEL // 8, 8, C), jnp.float32),
            pltpu.VMEM((D_MODEL // 8, 8, C), jnp.float32),
            pltpu.SemaphoreType.DMA,
            pltpu.SemaphoreType.DMA,
            pltpu.SemaphoreType.DMA,
            pltpu.SemaphoreType.DMA,
            pltpu.SemaphoreType.DMA,
            pltpu.SemaphoreType.DMA,
            pltpu.SemaphoreType.DMA,
            pltpu.SemaphoreType.DMA,
        ],
    )
    def k(lut_hbm, idx_hbm, out_hbm, idx_v,
          g0, g1, g2, g3, t0, t1, t2, t3,
          gs0, gs1, gs2, gs3, ss0, ss1, ss2, ss3):
        gbuf = (g0, g1, g2, g3)
        tbuf = (t0, t1, t2, t3)
        gsem = (gs0, gs1, gs2, gs3)
        ssem = (ss0, ss1, ss2, ss3)
        wid = lax.axis_index("s") * NC + lax.axis_index("c")
        chunk0 = wid * T
        pltpu.sync_copy(idx_hbm.at[pl.ds(chunk0, T)], idx_v)

        iota = lax.iota(jnp.int32, 16)
        rows_lb = [iota + (lb * 16) for lb in range(8)]

        def start_gather(j, slot):
            pltpu.async_copy(lut_hbm.at[idx_v.at[j]], gbuf[slot], gsem[slot])

        def wait_gather(j, slot):
            pltpu.make_async_copy(
                lut_hbm.at[idx_v.at[j]], gbuf[slot], gsem[slot]).wait()

        def transpose_scale(slot):
            src = gbuf[slot]
            dst = tbuf[slot]

            @plsc.parallel_loop(0, 8, 1)
            def _(dt):
                for sub in range(8):
                    d = dt * 8 + sub
                    col = jnp.full((16,), 0, jnp.int32) + d
                    for lb in range(8):
                        v = plsc.load_gather(src, [rows_lb[lb], col])
                        dst[dt, sub, pl.ds(lb * 16, 16)] = v * SCALE

        def out_slice(j):
            c = chunk0 + j
            s = c // n_bt
            bt = lax.rem(c, n_bt)
            return out_hbm.at[s, :, bt, :, :]

        def start_scatter(j, slot):
            pltpu.async_copy(tbuf[slot], out_slice(j), ssem[slot])

        def wait_scatter(j, slot):
            pltpu.make_async_copy(tbuf[slot], out_slice(j), ssem[slot]).wait()

        start_gather(0, 0)
        start_gather(1, 1)

        def group_body(g, carry):
            j0 = g * 4
            for off in range(4):
                j = j0 + off
                wait_gather(j, off)
                pl.when(j >= 4)(lambda: wait_scatter(j - 4, off))
                transpose_scale(off)
                start_scatter(j, off)
                pl.when(j <= T - 3)(
                    lambda: start_gather(j + 2, (off + 2) % 4))
            return carry

        lax.fori_loop(0, groups, group_body, 0)

        for off in range(4):
            wait_scatter(T - 4 + off, off)

    return k


def kernel(x, lut):
    B, S = x.shape
    R = B * S
    xt = jnp.transpose(x).reshape(R // C, C)
    z = _build_sc_kernel(S, B)(lut, xt)
    out = jnp.transpose(z, (0, 2, 4, 1, 3)).reshape(S, B, D_MODEL)
    return out


# speedup vs baseline: 1.3657x; 1.3657x over previous
import functools

import jax
import jax.numpy as jnp
from jax import lax
from jax.experimental import pallas as pl
from jax.experimental.pallas import tpu as pltpu
from jax.experimental.pallas import tpu_sc as plsc

D_MODEL = 64
SCALE = 8.0
NC, NS = 2, 16
NW = NC * NS
C = 128


def _build_sc_kernel(S, B):
    R = S * B
    n_bt = B // C
    rows_per_w = R // NW
    T = rows_per_w // C
    assert T % 4 == 0 and T >= 8
    groups = T // 4
    mesh = plsc.VectorSubcoreMesh(core_axis_name="c", subcore_axis_name="s")

    @functools.partial(
        pl.kernel,
        out_type=jax.ShapeDtypeStruct((S, D_MODEL // 8, n_bt, 8 * C),
                                      jnp.float32),
        mesh=mesh,
        compiler_params=pltpu.CompilerParams(use_tc_tiling_on_sc=False,
                                             needs_layout_passes=False),
        scratch_types=[
            pltpu.VMEM((T, C), jnp.int32),
            pltpu.VMEM((C, D_MODEL), jnp.float32),
            pltpu.VMEM((C, D_MODEL), jnp.float32),
            pltpu.VMEM((C, D_MODEL), jnp.float32),
            pltpu.VMEM((C, D_MODEL), jnp.float32),
            pltpu.VMEM((D_MODEL * C,), jnp.float32),
            pltpu.VMEM((D_MODEL * C,), jnp.float32),
            pltpu.VMEM((D_MODEL * C,), jnp.float32),
            pltpu.VMEM((D_MODEL * C,), jnp.float32),
            pltpu.SemaphoreType.DMA,
            pltpu.SemaphoreType.DMA,
            pltpu.SemaphoreType.DMA,
            pltpu.SemaphoreType.DMA,
            pltpu.SemaphoreType.DMA,
            pltpu.SemaphoreType.DMA,
            pltpu.SemaphoreType.DMA,
            pltpu.SemaphoreType.DMA,
        ],
    )
    def k(lut_hbm, idx_hbm, out_hbm, idx_v,
          g0, g1, g2, g3, t0, t1, t2, t3,
          gs0, gs1, gs2, gs3, ss0, ss1, ss2, ss3):
        gbuf = (g0, g1, g2, g3)
        tbuf = (t0, t1, t2, t3)
        gsem = (gs0, gs1, gs2, gs3)
        ssem = (ss0, ss1, ss2, ss3)
        wid = lax.axis_index("s") * NC + lax.axis_index("c")
        chunk0 = wid * T
        pltpu.sync_copy(idx_hbm.at[pl.ds(chunk0, T)], idx_v)

        iota = lax.iota(jnp.int32, 16)
        diag = [lax.rem(iota + g, 16) for g in range(16)]
        sg = [diag[g] * 128 + iota for g in range(16)]

        def start_gather(j, slot):
            pltpu.async_copy(lut_hbm.at[idx_v.at[j]], gbuf[slot], gsem[slot])

        def wait_gather(j, slot):
            pltpu.make_async_copy(
                lut_hbm.at[idx_v.at[j]], gbuf[slot], gsem[slot]).wait()

        def transpose_scale(slot):
            src = gbuf[slot]
            dst = tbuf[slot]

            @plsc.parallel_loop(0, C // 16, 1)
            def _(i):
                b0 = i * 16
                rowv = iota + b0
                for d0 in range(0, D_MODEL, 16):
                    for g in range(16):
                        v = plsc.load_gather(src, [rowv, diag[g] + d0])
                        plsc.store_scatter(
                            dst, [sg[g] + (d0 * 128 + b0)], v * SCALE)

        def start_scatter(j, slot):
            c = chunk0 + j
            s = c // n_bt
            bt = lax.rem(c, n_bt)
            for dt in range(D_MODEL // 8):
                pltpu.async_copy(
                    tbuf[slot].at[pl.ds(dt * 8 * C, 8 * C)],
                    out_hbm.at[s, dt, bt, :], ssem[slot])

        def wait_scatter(j, slot):
            c = chunk0 + j
            s = c // n_bt
            bt = lax.rem(c, n_bt)
            for dt in range(D_MODEL // 8):
                pltpu.make_async_copy(
                    tbuf[slot].at[pl.ds(dt * 8 * C, 8 * C)],
                    out_hbm.at[s, dt, bt, :], ssem[slot]).wait()

        start_gather(0, 0)
        start_gather(1, 1)

        def group_body(g, carry):
            j0 = g * 4
            for off in range(4):
                j = j0 + off
                wait_gather(j, off)
                pl.when(j >= 4)(lambda: wait_scatter(j - 4, off))
                transpose_scale(off)
                start_scatter(j, off)
                pl.when(j <= T - 3)(
                    lambda: start_gather(j + 2, (off + 2) % 4))
            return carry

        lax.fori_loop(0, groups, group_body, 0)

        for off in range(4):
            wait_scatter(T - 4 + off, off)

    return k


def kernel(x, lut):
    B, S = x.shape
    R = B * S
    xt = jnp.transpose(x).reshape(R // C, C)
    t2 = lax.optimization_barrier(jnp.reshape(lut, (lut.shape[0] // 2, 128)))
    table = jnp.reshape(t2, lut.shape)
    z = _build_sc_kernel(S, B)(table, xt)
    z5 = z.reshape(S, D_MODEL // 8, B // C, 8, C)
    out = jnp.transpose(z5, (0, 2, 4, 1, 3)).reshape(S, B, D_MODEL)
    return out


# speedup vs baseline: 1.3727x; 1.0051x over previous
import functools

import jax
import jax.numpy as jnp
from jax import lax
from jax.experimental import pallas as pl
from jax.experimental.pallas import tpu as pltpu
from jax.experimental.pallas import tpu_sc as plsc

D_MODEL = 64
SCALE = 8.0
NC, NS = 2, 16
NW = NC * NS
C = 128


def _build_sc_kernel(S, B):
    R = S * B
    n_bt = B // C
    rows_per_w = R // NW
    T = rows_per_w // C
    assert T % 4 == 0 and T >= 8
    groups = T // 4
    mesh = plsc.VectorSubcoreMesh(core_axis_name="c", subcore_axis_name="s")

    @functools.partial(
        pl.kernel,
        out_type=jax.ShapeDtypeStruct((S, D_MODEL // 8, n_bt, 8 * C),
                                      jnp.float32),
        mesh=mesh,
        compiler_params=pltpu.CompilerParams(use_tc_tiling_on_sc=False,
                                             needs_layout_passes=False),
        scratch_types=[
            pltpu.VMEM((T, C), jnp.int32),
            pltpu.VMEM((C, D_MODEL), jnp.float32),
            pltpu.VMEM((C, D_MODEL), jnp.float32),
            pltpu.VMEM((C, D_MODEL), jnp.float32),
            pltpu.VMEM((C, D_MODEL), jnp.float32),
            pltpu.VMEM((D_MODEL * C,), jnp.float32),
            pltpu.VMEM((D_MODEL * C,), jnp.float32),
            pltpu.VMEM((D_MODEL * C,), jnp.float32),
            pltpu.VMEM((D_MODEL * C,), jnp.float32),
            pltpu.SemaphoreType.DMA,
            pltpu.SemaphoreType.DMA,
            pltpu.SemaphoreType.DMA,
            pltpu.SemaphoreType.DMA,
            pltpu.SemaphoreType.DMA,
            pltpu.SemaphoreType.DMA,
            pltpu.SemaphoreType.DMA,
            pltpu.SemaphoreType.DMA,
        ],
    )
    def k(lut_hbm, idx_hbm, out_hbm, idx_v,
          g0, g1, g2, g3, t0, t1, t2, t3,
          gs0, gs1, gs2, gs3, ss0, ss1, ss2, ss3):
        gbuf = (g0, g1, g2, g3)
        tbuf = (t0, t1, t2, t3)
        gsem = (gs0, gs1, gs2, gs3)
        ssem = (ss0, ss1, ss2, ss3)
        wid = lax.axis_index("s") * NC + lax.axis_index("c")
        chunk0 = wid * T
        pltpu.sync_copy(idx_hbm.at[pl.ds(chunk0, T)], idx_v)

        iota = lax.iota(jnp.int32, 16)
        diag = [lax.rem(iota + g, 16) for g in range(16)]

        def start_gather(j, slot):
            pltpu.async_copy(lut_hbm.at[idx_v.at[j]], gbuf[slot], gsem[slot])

        def wait_gather(j, slot):
            pltpu.make_async_copy(
                lut_hbm.at[idx_v.at[j]], gbuf[slot], gsem[slot]).wait()

        def transpose_scale(slot):
            src = gbuf[slot]
            dst = tbuf[slot]

            @plsc.parallel_loop(0, C // 16, 1)
            def _(i):
                b0 = i * 16
                rowv = iota + b0
                for d0 in range(0, D_MODEL, 16):
                    sbase = iota + (d0 * C + b0)
                    for g in range(16):
                        colv = diag[g] + d0
                        v = plsc.load_gather(src, [rowv, colv])
                        sidx = (diag[g] << 7) + sbase
                        plsc.store_scatter(dst, [sidx], v * SCALE)

        def start_scatter(j, slot):
            c = chunk0 + j
            s = c // n_bt
            bt = lax.rem(c, n_bt)
            for dt in range(D_MODEL // 8):
                pltpu.async_copy(
                    tbuf[slot].at[pl.ds(dt * 8 * C, 8 * C)],
                    out_hbm.at[s, dt, bt, :], ssem[slot])

        def wait_scatter(j, slot):
            c = chunk0 + j
            s = c // n_bt
            bt = lax.rem(c, n_bt)
            for dt in range(D_MODEL // 8):
                pltpu.make_async_copy(
                    tbuf[slot].at[pl.ds(dt * 8 * C, 8 * C)],
                    out_hbm.at[s, dt, bt, :], ssem[slot]).wait()

        start_gather(0, 0)
        start_gather(1, 1)

        def group_body(g, carry):
            j0 = g * 4
            for off in range(4):
                j = j0 + off
                wait_gather(j, off)
                pl.when(j >= 4)(lambda: wait_scatter(j - 4, off))
                transpose_scale(off)
                start_scatter(j, off)
                pl.when(j <= T - 3)(
                    lambda: start_gather(j + 2, (off + 2) % 4))
            return carry

        lax.fori_loop(0, groups, group_body, 0)

        for off in range(4):
            wait_scatter(T - 4 + off, off)

    return k


def kernel(x, lut):
    B, S = x.shape
    R = B * S
    xt = jnp.transpose(x).reshape(R // C, C)
    t2 = lax.optimization_barrier(jnp.reshape(lut, (lut.shape[0] // 2, 128)))
    table = jnp.reshape(t2, lut.shape)
    z = _build_sc_kernel(S, B)(table, xt)
    z5 = z.reshape(S, D_MODEL // 8, B // C, 8, C)
    out = jnp.transpose(z5, (0, 2, 4, 1, 3)).reshape(S, B, D_MODEL)
    return out


# speedup vs baseline: 1.6660x; 1.2137x over previous
import functools

import jax
import jax.numpy as jnp
from jax import lax
from jax.experimental import pallas as pl
from jax.experimental.pallas import tpu as pltpu
from jax.experimental.pallas import tpu_sc as plsc

D_MODEL = 64
SCALE = 8.0
NC, NS = 2, 16
NW = NC * NS
C = 128


def _build_converter(V):
    full_blocks = (V // C) * C // C
    per_w = full_blocks // NW
    rem = full_blocks - per_w * NW
    assert per_w % 4 == 0
    groups = per_w // 4
    tail = V - full_blocks * C
    mesh = plsc.VectorSubcoreMesh(core_axis_name="c", subcore_axis_name="s")

    @functools.partial(
        pl.kernel,
        out_type=jax.ShapeDtypeStruct((V * D_MODEL,), jnp.float32),
        mesh=mesh,
        compiler_params=pltpu.CompilerParams(needs_layout_passes=False),
        scratch_types=[
            pltpu.VMEM((D_MODEL, C), jnp.float32),
            pltpu.VMEM((D_MODEL, C), jnp.float32),
            pltpu.VMEM((D_MODEL, C), jnp.float32),
            pltpu.VMEM((D_MODEL, C), jnp.float32),
            pltpu.VMEM((C * D_MODEL,), jnp.float32),
            pltpu.VMEM((C * D_MODEL,), jnp.float32),
            pltpu.VMEM((C * D_MODEL,), jnp.float32),
            pltpu.VMEM((C * D_MODEL,), jnp.float32),
            pltpu.SemaphoreType.DMA,
            pltpu.SemaphoreType.DMA,
            pltpu.SemaphoreType.DMA,
            pltpu.SemaphoreType.DMA,
            pltpu.SemaphoreType.DMA,
            pltpu.SemaphoreType.DMA,
            pltpu.SemaphoreType.DMA,
            pltpu.SemaphoreType.DMA,
        ],
    )
    def k(lutT_hbm, tail_hbm, out_hbm, i0, i1, i2, i3, o0, o1, o2, o3,
          gi0, gi1, gi2, gi3, so0, so1, so2, so3):
        ibuf = (i0, i1, i2, i3)
        obuf = (o0, o1, o2, o3)
        isem = (gi0, gi1, gi2, gi3)
        osem = (so0, so1, so2, so3)
        wid = lax.axis_index("s") * NC + lax.axis_index("c")
        base_w = wid * per_w

        iota = lax.iota(jnp.int32, 16)
        i64 = iota * D_MODEL
        diag = [lax.rem(iota + g, 16) for g in range(16)]

        def start_in(k_, slot):
            blk = base_w + k_
            pltpu.async_copy(lutT_hbm.at[:, pl.ds(blk * C, C)],
                             ibuf[slot], isem[slot])

        def wait_in(k_, slot):
            blk = base_w + k_
            pltpu.make_async_copy(lutT_hbm.at[:, pl.ds(blk * C, C)],
                                  ibuf[slot], isem[slot]).wait()

        def transpose(slot):
            src = ibuf[slot]
            dst = obuf[slot]

            @plsc.parallel_loop(0, C // 16, 1)
            def _(i):
                r0 = i * 16
                colv = iota + r0
                for d0 in range(0, D_MODEL, 16):
                    sb = i64 + (r0 * D_MODEL + d0)
                    for g in range(16):
                        v = plsc.load_gather(src, [diag[g] + d0, colv])
                        plsc.store_scatter(dst, [diag[g] + sb], v)

        def start_out(k_, slot):
            blk = base_w + k_
            pltpu.async_copy(obuf[slot],
                             out_hbm.at[pl.ds(blk * C * D_MODEL,
                                              C * D_MODEL)], osem[slot])

        def wait_out(k_, slot):
            blk = base_w + k_
            pltpu.make_async_copy(obuf[slot],
                                  out_hbm.at[pl.ds(blk * C * D_MODEL,
                                                   C * D_MODEL)],
                                  osem[slot]).wait()

        start_in(0, 0)
        start_in(1, 1)

        def group_body(g, carry):
            j0 = g * 4
            for off in range(4):
                k_ = j0 + off
                wait_in(k_, off)
                pl.when(k_ >= 4)(lambda: wait_out(k_ - 4, off))
                transpose(off)
                start_out(k_, off)
                pl.when(k_ <= per_w - 3)(
                    lambda: start_in(k_ + 2, (off + 2) % 4))
            return carry

        lax.fori_loop(0, groups, group_body, 0)
        for off in range(4):
            wait_out(per_w - 4 + off, off)

        @pl.when(wid < rem)
        def _():
            blk = NW * per_w + wid
            pltpu.async_copy(lutT_hbm.at[:, pl.ds(blk * C, C)],
                             ibuf[0], isem[0]).wait()
            transpose(0)
            pltpu.async_copy(
                obuf[0],
                out_hbm.at[pl.ds(blk * C * D_MODEL, C * D_MODEL)],
                osem[0]).wait()

        @pl.when(wid == rem)
        def _():
            pltpu.async_copy(tail_hbm, obuf[1].at[pl.ds(0, tail * D_MODEL)],
                             isem[1]).wait()
            pltpu.async_copy(
                obuf[1].at[pl.ds(0, tail * D_MODEL)],
                out_hbm.at[pl.ds(full_blocks * C * D_MODEL, tail * D_MODEL)],
                osem[1]).wait()

    return k


def _build_sc_kernel(S, B):
    R = S * B
    n_bt = B // C
    rows_per_w = R // NW
    T = rows_per_w // C
    assert T % 4 == 0 and T >= 8
    groups = T // 4
    mesh = plsc.VectorSubcoreMesh(core_axis_name="c", subcore_axis_name="s")

    @functools.partial(
        pl.kernel,
        out_type=jax.ShapeDtypeStruct((S, D_MODEL // 8, n_bt, 8 * C),
                                      jnp.float32),
        mesh=mesh,
        compiler_params=pltpu.CompilerParams(use_tc_tiling_on_sc=False,
                                             needs_layout_passes=False),
        scratch_types=[
            pltpu.VMEM((T, C), jnp.int32),
            pltpu.VMEM((C, D_MODEL), jnp.float32),
            pltpu.VMEM((C, D_MODEL), jnp.float32),
            pltpu.VMEM((C, D_MODEL), jnp.float32),
            pltpu.VMEM((C, D_MODEL), jnp.float32),
            pltpu.VMEM((D_MODEL * C,), jnp.float32),
            pltpu.VMEM((D_MODEL * C,), jnp.float32),
            pltpu.VMEM((D_MODEL * C,), jnp.float32),
            pltpu.VMEM((D_MODEL * C,), jnp.float32),
            pltpu.SemaphoreType.DMA,
            pltpu.SemaphoreType.DMA,
            pltpu.SemaphoreType.DMA,
            pltpu.SemaphoreType.DMA,
            pltpu.SemaphoreType.DMA,
            pltpu.SemaphoreType.DMA,
            pltpu.SemaphoreType.DMA,
            pltpu.SemaphoreType.DMA,
        ],
    )
    def k(lut_hbm, idx_hbm, out_hbm, idx_v,
          g0, g1, g2, g3, t0, t1, t2, t3,
          gs0, gs1, gs2, gs3, ss0, ss1, ss2, ss3):
        gbuf = (g0, g1, g2, g3)
        tbuf = (t0, t1, t2, t3)
        gsem = (gs0, gs1, gs2, gs3)
        ssem = (ss0, ss1, ss2, ss3)
        wid = lax.axis_index("s") * NC + lax.axis_index("c")
        chunk0 = wid * T
        pltpu.sync_copy(idx_hbm.at[pl.ds(chunk0, T)], idx_v)

        iota = lax.iota(jnp.int32, 16)
        diag = [lax.rem(iota + g, 16) for g in range(16)]

        def start_gather(j, slot):
            pltpu.async_copy(lut_hbm.at[idx_v.at[j]], gbuf[slot], gsem[slot])

        def wait_gather(j, slot):
            pltpu.make_async_copy(
                lut_hbm.at[idx_v.at[j]], gbuf[slot], gsem[slot]).wait()

        def transpose_scale(slot):
            src = gbuf[slot]
            dst = tbuf[slot]

            @plsc.parallel_loop(0, C // 16, 1)
            def _(i):
                b0 = i * 16
                rowv = iota + b0
                for d0 in range(0, D_MODEL, 16):
                    sbase = iota + (d0 * C + b0)
                    for g in range(16):
                        colv = diag[g] + d0
                        v = plsc.load_gather(src, [rowv, colv])
                        sidx = (diag[g] << 7) + sbase
                        plsc.store_scatter(dst, [sidx], v * SCALE)

        def start_scatter(j, slot):
            c = chunk0 + j
            s = c // n_bt
            bt = lax.rem(c, n_bt)
            for dt in range(D_MODEL // 8):
                pltpu.async_copy(
                    tbuf[slot].at[pl.ds(dt * 8 * C, 8 * C)],
                    out_hbm.at[s, dt, bt, :], ssem[slot])

        def wait_scatter(j, slot):
            c = chunk0 + j
            s = c // n_bt
            bt = lax.rem(c, n_bt)
            for dt in range(D_MODEL // 8):
                pltpu.make_async_copy(
                    tbuf[slot].at[pl.ds(dt * 8 * C, 8 * C)],
                    out_hbm.at[s, dt, bt, :], ssem[slot]).wait()

        start_gather(0, 0)
        start_gather(1, 1)

        def group_body(g, carry):
            j0 = g * 4
            for off in range(4):
                j = j0 + off
                wait_gather(j, off)
                pl.when(j >= 4)(lambda: wait_scatter(j - 4, off))
                transpose_scale(off)
                start_scatter(j, off)
                pl.when(j <= T - 3)(
                    lambda: start_gather(j + 2, (off + 2) % 4))
            return carry

        lax.fori_loop(0, groups, group_body, 0)

        for off in range(4):
            wait_scatter(T - 4 + off, off)

    return k


def kernel(x, lut):
    B, S = x.shape
    R = B * S
    xt = jnp.transpose(x).reshape(R // C, C)
    V = lut.shape[0]
    lutT = jnp.transpose(lut)
    tail_flat = lut[(V // C) * C:, :].reshape(-1)
    cvt = _build_converter(V)(lutT, tail_flat)
    table = cvt.reshape(V, D_MODEL)
    z = _build_sc_kernel(S, B)(table, xt)
    z5 = z.reshape(S, D_MODEL // 8, B // C, 8, C)
    out = jnp.transpose(z5, (0, 2, 4, 1, 3)).reshape(S, B, D_MODEL)
    return out
